# baseline (speedup 1.0000x reference)
import functools

import jax
import jax.numpy as jnp
from jax import lax
from jax.experimental import pallas as pl
from jax.experimental.pallas import tpu as pltpu
from jax.experimental.pallas import tpu_sc as plsc

_NUM_CORES = 2
_NUM_SUBCORES = 16
_NUM_WORKERS = _NUM_CORES * _NUM_SUBCORES
_IDX_CHUNK = 128


def _sc_gather_pad(theta_pad, idx2d, batch):
    dim = 128
    b_per_w = batch // _NUM_WORKERS
    chunks = b_per_w // _IDX_CHUNK
    mesh = plsc.VectorSubcoreMesh(core_axis_name="c", subcore_axis_name="s")

    @functools.partial(
        pl.kernel,
        mesh=mesh,
        out_type=jax.ShapeDtypeStruct((batch, dim), jnp.float32),
        scratch_types=[
            pltpu.VMEM((chunks, _IDX_CHUNK), jnp.int32),
            pltpu.VMEM((b_per_w, dim), jnp.float32),
            pltpu.SemaphoreType.DMA,
        ],
    )
    def gather_kernel(theta_hbm, idx_hbm, out_hbm, idx_v, rows_v, sem):
        wid = lax.axis_index("s") * _NUM_CORES + lax.axis_index("c")
        base = wid * b_per_w
        pltpu.sync_copy(idx_hbm.at[pl.ds(wid * chunks, chunks)], idx_v)
        copies = []
        for j in range(chunks):
            copies.append(
                pltpu.async_copy(
                    theta_hbm.at[idx_v.at[j]],
                    rows_v.at[pl.ds(j * _IDX_CHUNK, _IDX_CHUNK)],
                    sem,
                )
            )
        for c in copies:
            c.wait()
        pltpu.sync_copy(rows_v, out_hbm.at[pl.ds(base, b_per_w)])

    return gather_kernel(theta_pad, idx2d)


def _tc_utility_logsoftmax(theta, alpha_item, batch, num_items, dim):
    blk = 2048

    def body(theta_ref, alpha_ref, out_ref):
        th = theta_ref[...]
        al = alpha_ref[...]
        u = lax.dot_general(
            th, al, (((1,), (1,)), ((), ())), preferred_element_type=jnp.float32
        )
        m = jnp.max(u, axis=-1, keepdims=True)
        e = jnp.exp(u - m)
        s = jnp.sum(e, axis=-1, keepdims=True)
        out_ref[...] = u - m - jnp.log(s)

    return pl.pallas_call(
        body,
        grid=(batch // blk,),
        in_specs=[
            pl.BlockSpec((blk, dim), lambda i: (i, 0)),
            pl.BlockSpec((num_items, dim), lambda i: (0, 0)),
        ],
        out_specs=pl.BlockSpec((blk, num_items), lambda i: (i, 0)),
        out_shape=jax.ShapeDtypeStruct((batch, num_items), jnp.float32),
    )(theta, alpha_item)


def kernel(user_index, theta_user, alpha_item):
    batch = user_index.shape[0]
    num_items, dim = alpha_item.shape
    idx2d = user_index.astype(jnp.int32).reshape(batch // _IDX_CHUNK, _IDX_CHUNK)
    theta_pad = jnp.pad(theta_user, ((0, 0), (0, 128 - dim)))
    alpha_pad = jnp.pad(alpha_item, ((0, 0), (0, 128 - dim)))
    theta = _sc_gather_pad(theta_pad, idx2d, batch)
    return _tc_utility_logsoftmax(theta, alpha_pad, batch, num_items, 128)
